# baseline (speedup 1.0000x reference)
import functools

import jax
import jax.numpy as jnp
from jax import lax
from jax.experimental import pallas as pl
from jax.experimental.pallas import tpu as pltpu
from jax.experimental.pallas import tpu_sc as plsc

_BATCH, _BEAM, _VOCAB = 64, 8, 100000
_ROWS = _BATCH * _BEAM
_BLK = 160
_NBLK = _VOCAB // _BLK
_K = 16
_SLABS = 5
_CHUNK = _SLABS * _BLK
_STEPS = _VOCAB // _CHUNK
_LANES = 32
_IDXW = 16
_TROWS = _VOCAB * _ROWS // _LANES
_BIGI = 2 ** 30


def _stage_a(x_ref, bm_ref, lse_ref, m_ref, s_ref):
    i = pl.program_id(0)
    x = x_ref[...]
    v = lax.broadcasted_iota(jnp.int32, x.shape, 0) + i * _CHUNK
    x = jnp.where(v < 3, -jnp.inf, x)
    bm5 = jnp.max(x.reshape(_SLABS, _BLK, _ROWS), axis=1)
    bm_ref[...] = bm5.reshape(1, _SLABS, _ROWS)
    cmax = jnp.max(bm5, axis=0, keepdims=True)

    @pl.when(i == 0)
    def _():
        m_ref[...] = jnp.full((1, _ROWS), -jnp.inf, jnp.float32)
        s_ref[...] = jnp.zeros((1, _ROWS), jnp.float32)

    mo = m_ref[...]
    mn = jnp.maximum(mo, cmax)
    s_ref[...] = s_ref[...] * jnp.exp(mo - mn) + jnp.sum(
        jnp.exp(x - mn), axis=0, keepdims=True)
    m_ref[...] = mn

    @pl.when(i == _STEPS - 1)
    def _():
        lse_ref[...] = m_ref[...] + jnp.log(s_ref[...])


def _stage_b(bm_ref, lse_ref, sel_ref, ids_ref, gidx_ref, lsel_ref):
    bm = bm_ref[...]
    lse = lse_ref[...]
    sel = sel_ref[...]
    work = (bm - lse[:, :, None]) + sel[:, :, None]
    fid = (lax.broadcasted_iota(jnp.int32, work.shape, 1) * _NBLK
           + lax.broadcasted_iota(jnp.int32, work.shape, 2))
    cols = []
    for _ in range(_K):
        mx = jnp.max(work, axis=(1, 2), keepdims=True)
        sid = jnp.min(jnp.where(work == mx, fid, _BIGI), axis=(1, 2), keepdims=True)
        cols.append(sid[:, :, 0])
        work = jnp.where(fid == sid, -jnp.inf, work)
    ids = jnp.concatenate(cols, axis=1)
    ids_ref[...] = ids
    jv = ids // _NBLK
    cv = ids - jv * _NBLK
    bvec = lax.broadcasted_iota(jnp.int32, (_BATCH, _K), 0)
    q = (bvec * _BEAM + jv) // _LANES
    t = lax.broadcasted_iota(jnp.int32, (_BATCH, _K, _BLK), 2)
    gidx_ref[...] = ((cv[:, :, None] * _BLK + t) * (_ROWS // _LANES)
                     + q[:, :, None])
    lsel = (bvec * _BEAM + jv) % _LANES
    lsel_ref[...] = jnp.broadcast_to(lsel[:, :, None], (_BATCH, _K, _IDXW))


_sc_mesh = plsc.VectorSubcoreMesh(
    core_axis_name="c", subcore_axis_name="s", num_cores=2, num_subcores=16)


@functools.partial(
    pl.kernel,
    out_type=jax.ShapeDtypeStruct((_BATCH * _K, _BLK), jnp.float32),
    mesh=_sc_mesh,
    scratch_types=[
        pltpu.VMEM((2 * _K, _BLK), jnp.int32),
        pltpu.VMEM((2 * _K, _IDXW), jnp.int32),
        pltpu.VMEM((_K, _BLK, _LANES), jnp.float32),
        pltpu.VMEM((2 * _K, _BLK), jnp.float32),
        pltpu.SemaphoreType.DMA,
    ],
    compiler_params=pltpu.CompilerParams(
        use_tc_tiling_on_sc=False, needs_layout_passes=False),
)
def _sc_gather(gidx_hbm, lsel_hbm, table_hbm, out_hbm,
               idx_v, lsel_v, rows_v, comp_v, sem):
    w = lax.axis_index("s") * 2 + lax.axis_index("c")
    pltpu.sync_copy(gidx_hbm.at[pl.ds(w * 2 * _K, 2 * _K)], idx_v)
    pltpu.sync_copy(lsel_hbm.at[pl.ds(w * 2 * _K, 2 * _K)], lsel_v)
    half = _BLK // 2
    for u in range(2):
        base = u * _K
        for grp in range(2):
            copies = []
            for kk in range(8):
                k = grp * 8 + kk
                for h in range(2):
                    copies.append(pltpu.async_copy(
                        table_hbm.at[idx_v.at[base + k, pl.ds(h * half, half)]],
                        rows_v.at[k, pl.ds(h * half, half)], sem))
            for c in copies:
                c.wait()
        for k in range(_K):
            kvec = jnp.full((_IDXW,), k, jnp.int32)
            lanevec = lsel_v[base + k, :]
            for t in range(_BLK // _IDXW):
                tvec = t * _IDXW + lax.iota(jnp.int32, _IDXW)
                vals = plsc.load_gather(rows_v, [kvec, tvec, lanevec])
                comp_v[base + k, pl.ds(t * _IDXW, _IDXW)] = vals
    pltpu.sync_copy(comp_v, out_hbm.at[pl.ds(w * 2 * _K, 2 * _K)])


def _stage_d(cand_ref, ids_ref, lse_ref, sel_ref, sc_ref, wd_ref, bx_ref):
    rows = cand_ref[...]
    ids = ids_ref[...]
    lse = lse_ref[...]
    sel = sel_ref[...]
    jv = ids // _NBLK
    cv = ids - jv * _NBLK
    beams = lax.broadcasted_iota(jnp.int32, (_BATCH, _K, _BEAM), 2)
    onehot = jv[:, :, None] == beams
    lsek = jnp.sum(jnp.where(onehot, lse[:, None, :], 0.0), axis=2)
    selk = jnp.sum(jnp.where(onehot, sel[:, None, :], 0.0), axis=2)
    sc = (rows - lsek[:, :, None]) + selk[:, :, None]
    t = lax.broadcasted_iota(jnp.int32, (_BATCH, _K, _BLK), 2)
    word = cv[:, :, None] * _BLK + t
    sc = jnp.where(word < 3, -jnp.inf, sc)
    fpk = jv[:, :, None] * _VOCAB + word
    scs, pks = [], []
    for _ in range(_K):
        mx = jnp.max(sc, axis=(1, 2), keepdims=True)
        psel = jnp.min(jnp.where(sc == mx, fpk, _BIGI), axis=(1, 2), keepdims=True)
        scs.append(mx[:, :, 0])
        pks.append(psel[:, :, 0])
        sc = jnp.where(fpk == psel, -jnp.inf, sc)
    sc_ref[...] = jnp.concatenate(scs, axis=1)
    pk = jnp.concatenate(pks, axis=1)
    wd_ref[...] = pk % _VOCAB
    bx_ref[...] = pk // _VOCAB


def kernel(logits, scores, position):
    xt = logits.T
    table = xt.reshape(_TROWS, _LANES)
    pos = jnp.asarray(position, jnp.int32)
    sel8 = lax.dynamic_index_in_dim(scores, pos - 1, axis=2, keepdims=False)

    bm, lse = pl.pallas_call(
        _stage_a,
        grid=(_STEPS,),
        in_specs=[pl.BlockSpec((_CHUNK, _ROWS), lambda i: (i, 0))],
        out_specs=[
            pl.BlockSpec((1, _SLABS, _ROWS), lambda i: (i, 0, 0)),
            pl.BlockSpec((1, _ROWS), lambda i: (0, 0)),
        ],
        out_shape=[
            jax.ShapeDtypeStruct((_STEPS, _SLABS, _ROWS), jnp.float32),
            jax.ShapeDtypeStruct((1, _ROWS), jnp.float32),
        ],
        scratch_shapes=[
            pltpu.VMEM((1, _ROWS), jnp.float32),
            pltpu.VMEM((1, _ROWS), jnp.float32),
        ],
    )(xt)

    bmt = bm.reshape(_NBLK, _ROWS).T.reshape(_BATCH, _BEAM, _NBLK)
    lse8 = lse.reshape(_ROWS).reshape(_BATCH, _BEAM)

    ids, gidx, lsel = pl.pallas_call(
        _stage_b,
        out_shape=[
            jax.ShapeDtypeStruct((_BATCH, _K), jnp.int32),
            jax.ShapeDtypeStruct((_BATCH, _K, _BLK), jnp.int32),
            jax.ShapeDtypeStruct((_BATCH, _K, _IDXW), jnp.int32),
        ],
    )(bmt, lse8, sel8)

    cand = _sc_gather(gidx.reshape(_BATCH * _K, _BLK),
                      lsel.reshape(_BATCH * _K, _IDXW), table)

    scall, wd, bx = pl.pallas_call(
        _stage_d,
        out_shape=[
            jax.ShapeDtypeStruct((_BATCH, _K), jnp.float32),
            jax.ShapeDtypeStruct((_BATCH, _K), jnp.int32),
            jax.ShapeDtypeStruct((_BATCH, _K), jnp.int32),
        ],
    )(cand.reshape(_BATCH, _K, _BLK), ids, lse8, sel8)

    return (scall[:, :8], wd[:, :8], bx[:, :8], wd, bx)


# speedup vs baseline: 1.4120x; 1.4120x over previous
import functools

import jax
import jax.numpy as jnp
from jax import lax
from jax.experimental import pallas as pl
from jax.experimental.pallas import tpu as pltpu
from jax.experimental.pallas import tpu_sc as plsc

_BATCH, _BEAM, _VOCAB = 64, 8, 100000
_ROWS = _BATCH * _BEAM
_BLK = 160
_NBLK = _VOCAB // _BLK
_K = 16
_SLABS = 5
_CHUNK = _SLABS * _BLK
_STEPS = _VOCAB // _CHUNK
_LANES = 32
_IDXW = 16
_TROWS = _VOCAB * _ROWS // _LANES
_BIGI = 2 ** 30


def _stage_a(x_ref, bm_ref, lse_ref, m_ref, s_ref):
    i = pl.program_id(0)
    x = x_ref[...]
    v = lax.broadcasted_iota(jnp.int32, x.shape, 0) + i * _CHUNK
    x = jnp.where(v < 3, -jnp.inf, x)
    bm5 = jnp.max(x.reshape(_SLABS, _BLK, _ROWS), axis=1)
    bm_ref[...] = bm5.reshape(1, _SLABS, _ROWS)
    cmax = jnp.max(bm5, axis=0, keepdims=True)

    @pl.when(i == 0)
    def _():
        m_ref[...] = jnp.full((1, _ROWS), -jnp.inf, jnp.float32)
        s_ref[...] = jnp.zeros((1, _ROWS), jnp.float32)

    mo = m_ref[...]
    mn = jnp.maximum(mo, cmax)
    s_ref[...] = s_ref[...] * jnp.exp(mo - mn) + jnp.sum(
        jnp.exp(x - mn), axis=0, keepdims=True)
    m_ref[...] = mn

    @pl.when(i == _STEPS - 1)
    def _():
        lse_ref[...] = m_ref[...] + jnp.log(s_ref[...])


def _stage_b(bm_ref, lse_ref, sel_ref, ids_ref, gidx_ref, lsel_ref):
    bm = bm_ref[...]
    lse = lse_ref[...]
    sel = sel_ref[...]
    work = (bm - lse[:, :, None]) + sel[:, :, None]
    fid = (lax.broadcasted_iota(jnp.int32, work.shape, 1) * _NBLK
           + lax.broadcasted_iota(jnp.int32, work.shape, 2))
    cols = []
    for _ in range(_K):
        mx = jnp.max(work, axis=(1, 2), keepdims=True)
        sid = jnp.min(jnp.where(work == mx, fid, _BIGI), axis=(1, 2), keepdims=True)
        cols.append(sid[:, :, 0])
        work = jnp.where(fid == sid, -jnp.inf, work)
    ids = jnp.concatenate(cols, axis=1)
    ids_ref[...] = ids
    jv = ids // _NBLK
    cv = ids - jv * _NBLK
    bvec = lax.broadcasted_iota(jnp.int32, (_BATCH, _K), 0)
    q = (bvec * _BEAM + jv) // _LANES
    t = lax.broadcasted_iota(jnp.int32, (_BATCH, _K, _BLK), 2)
    g = ((cv[:, :, None] * _BLK + t) * (_ROWS // _LANES)
         + q[:, :, None])
    g1 = g // 128
    row128 = (g % 128) // 4
    r1 = row128 // 4
    r2 = row128 % 4
    r4 = g % 4
    gidx_ref[...] = ((g1 * 4 + r2) * 8 + r1) * 4 + r4
    lsel = (bvec * _BEAM + jv) % _LANES
    lsel_ref[...] = jnp.broadcast_to(lsel[:, :, None], (_BATCH, _K, _IDXW))


_sc_mesh = plsc.VectorSubcoreMesh(
    core_axis_name="c", subcore_axis_name="s", num_cores=2, num_subcores=16)


@functools.partial(
    pl.kernel,
    out_type=jax.ShapeDtypeStruct((_BATCH * _K, _BLK), jnp.float32),
    mesh=_sc_mesh,
    scratch_types=[
        pltpu.VMEM((2 * _K, _BLK), jnp.int32),
        pltpu.VMEM((2 * _K, _IDXW), jnp.int32),
        pltpu.VMEM((_K, _BLK, _LANES), jnp.float32),
        pltpu.VMEM((2 * _K, _BLK), jnp.float32),
        pltpu.SemaphoreType.DMA,
    ],
    compiler_params=pltpu.CompilerParams(
        use_tc_tiling_on_sc=False, needs_layout_passes=False),
)
def _sc_gather(gidx_hbm, lsel_hbm, table_hbm, out_hbm,
               idx_v, lsel_v, rows_v, comp_v, sem):
    w = lax.axis_index("s") * 2 + lax.axis_index("c")
    pltpu.sync_copy(gidx_hbm.at[pl.ds(w * 2 * _K, 2 * _K)], idx_v)
    pltpu.sync_copy(lsel_hbm.at[pl.ds(w * 2 * _K, 2 * _K)], lsel_v)
    half = _BLK // 2
    for u in range(2):
        base = u * _K
        for grp in range(2):
            copies = []
            for kk in range(8):
                k = grp * 8 + kk
                for h in range(2):
                    copies.append(pltpu.async_copy(
                        table_hbm.at[idx_v.at[base + k, pl.ds(h * half, half)]],
                        rows_v.at[k, pl.ds(h * half, half)], sem))
            for c in copies:
                c.wait()
        for k in range(_K):
            kvec = jnp.full((_IDXW,), k, jnp.int32)
            lanevec = lsel_v[base + k, :]
            for t in range(_BLK // _IDXW):
                tvec = t * _IDXW + lax.iota(jnp.int32, _IDXW)
                vals = plsc.load_gather(rows_v, [kvec, tvec, lanevec])
                comp_v[base + k, pl.ds(t * _IDXW, _IDXW)] = vals
    pltpu.sync_copy(comp_v, out_hbm.at[pl.ds(w * 2 * _K, 2 * _K)])


def _stage_d(cand_ref, ids_ref, lse_ref, sel_ref, sc_ref, wd_ref, bx_ref):
    rows = cand_ref[...]
    ids = ids_ref[...]
    lse = lse_ref[...]
    sel = sel_ref[...]
    jv = ids // _NBLK
    cv = ids - jv * _NBLK
    beams = lax.broadcasted_iota(jnp.int32, (_BATCH, _K, _BEAM), 2)
    onehot = jv[:, :, None] == beams
    lsek = jnp.sum(jnp.where(onehot, lse[:, None, :], 0.0), axis=2)
    selk = jnp.sum(jnp.where(onehot, sel[:, None, :], 0.0), axis=2)
    sc = (rows - lsek[:, :, None]) + selk[:, :, None]
    t = lax.broadcasted_iota(jnp.int32, (_BATCH, _K, _BLK), 2)
    word = cv[:, :, None] * _BLK + t
    sc = jnp.where(word < 3, -jnp.inf, sc)
    fpk = jv[:, :, None] * _VOCAB + word
    scs, pks = [], []
    for _ in range(_K):
        mx = jnp.max(sc, axis=(1, 2), keepdims=True)
        psel = jnp.min(jnp.where(sc == mx, fpk, _BIGI), axis=(1, 2), keepdims=True)
        scs.append(mx[:, :, 0])
        pks.append(psel[:, :, 0])
        sc = jnp.where(fpk == psel, -jnp.inf, sc)
    sc_ref[...] = jnp.concatenate(scs, axis=1)
    pk = jnp.concatenate(pks, axis=1)
    wd_ref[...] = pk % _VOCAB
    bx_ref[...] = pk // _VOCAB


def kernel(logits, scores, position):
    xt = logits.T
    table = (xt.reshape(_TROWS * _LANES // 4096, 8, 4, 128)
             .transpose(0, 2, 1, 3).reshape(_TROWS, _LANES))
    pos = jnp.asarray(position, jnp.int32)
    sel8 = lax.dynamic_index_in_dim(scores, pos - 1, axis=2, keepdims=False)

    bm, lse = pl.pallas_call(
        _stage_a,
        grid=(_STEPS,),
        in_specs=[pl.BlockSpec((_CHUNK, _ROWS), lambda i: (i, 0))],
        out_specs=[
            pl.BlockSpec((1, _SLABS, _ROWS), lambda i: (i, 0, 0)),
            pl.BlockSpec((1, _ROWS), lambda i: (0, 0)),
        ],
        out_shape=[
            jax.ShapeDtypeStruct((_STEPS, _SLABS, _ROWS), jnp.float32),
            jax.ShapeDtypeStruct((1, _ROWS), jnp.float32),
        ],
        scratch_shapes=[
            pltpu.VMEM((1, _ROWS), jnp.float32),
            pltpu.VMEM((1, _ROWS), jnp.float32),
        ],
    )(xt)

    bmt = bm.reshape(_NBLK, _ROWS).T.reshape(_BATCH, _BEAM, _NBLK)
    lse8 = lse.reshape(_ROWS).reshape(_BATCH, _BEAM)

    ids, gidx, lsel = pl.pallas_call(
        _stage_b,
        out_shape=[
            jax.ShapeDtypeStruct((_BATCH, _K), jnp.int32),
            jax.ShapeDtypeStruct((_BATCH, _K, _BLK), jnp.int32),
            jax.ShapeDtypeStruct((_BATCH, _K, _IDXW), jnp.int32),
        ],
    )(bmt, lse8, sel8)

    cand = _sc_gather(gidx.reshape(_BATCH * _K, _BLK),
                      lsel.reshape(_BATCH * _K, _IDXW), table)

    scall, wd, bx = pl.pallas_call(
        _stage_d,
        out_shape=[
            jax.ShapeDtypeStruct((_BATCH, _K), jnp.float32),
            jax.ShapeDtypeStruct((_BATCH, _K), jnp.int32),
            jax.ShapeDtypeStruct((_BATCH, _K), jnp.int32),
        ],
    )(cand.reshape(_BATCH, _K, _BLK), ids, lse8, sel8)

    return (scall[:, :8], wd[:, :8], bx[:, :8], wd, bx)


# speedup vs baseline: 1.8619x; 1.3186x over previous
import functools

import jax
import jax.numpy as jnp
from jax import lax
from jax.experimental import pallas as pl
from jax.experimental.pallas import tpu as pltpu
from jax.experimental.pallas import tpu_sc as plsc

_BATCH, _BEAM, _VOCAB = 64, 8, 100000
_ROWS = _BATCH * _BEAM
_BLK = 160
_NBLK = _VOCAB // _BLK
_K = 16
_SLABS = 25
_CHUNK = _SLABS * _BLK
_STEPS = _VOCAB // _CHUNK
_LANES = 32
_IDXW = 16
_TROWS = _VOCAB * _ROWS // _LANES
_BIGI = 2 ** 30


def _stage_a(x_ref, bm_ref, lse_ref, m_ref, s_ref):
    i = pl.program_id(0)
    x = x_ref[...]
    v = lax.broadcasted_iota(jnp.int32, x.shape, 0) + i * _CHUNK
    x = jnp.where(v < 3, -jnp.inf, x)
    bm5 = jnp.max(x.reshape(_SLABS, _BLK, _ROWS), axis=1)
    bm_ref[...] = bm5.reshape(1, _SLABS, _ROWS)
    cmax = jnp.max(bm5, axis=0, keepdims=True)

    @pl.when(i == 0)
    def _():
        m_ref[...] = jnp.full((1, _ROWS), -jnp.inf, jnp.float32)
        s_ref[...] = jnp.zeros((1, _ROWS), jnp.float32)

    mo = m_ref[...]
    mn = jnp.maximum(mo, cmax)
    s_ref[...] = s_ref[...] * jnp.exp(mo - mn) + jnp.sum(
        jnp.exp(x - mn), axis=0, keepdims=True)
    m_ref[...] = mn

    @pl.when(i == _STEPS - 1)
    def _():
        lse_ref[...] = m_ref[...] + jnp.log(s_ref[...])


def _stage_b(bm_ref, lse_ref, sel_ref, ids_ref, gidx_ref, lsel_ref):
    bm = bm_ref[...]
    lse = lse_ref[...]
    sel = sel_ref[...]
    work = (bm - lse[:, :, None]) + sel[:, :, None]
    fid = (lax.broadcasted_iota(jnp.int32, work.shape, 1) * _NBLK
           + lax.broadcasted_iota(jnp.int32, work.shape, 2))
    cols = []
    for _ in range(_K):
        mx = jnp.max(work, axis=(1, 2), keepdims=True)
        sid = jnp.min(jnp.where(work == mx, fid, _BIGI), axis=(1, 2), keepdims=True)
        cols.append(sid[:, :, 0])
        work = jnp.where(fid == sid, -jnp.inf, work)
    ids = jnp.concatenate(cols, axis=1)
    ids_ref[...] = ids
    jv = ids // _NBLK
    cv = ids - jv * _NBLK
    bvec = lax.broadcasted_iota(jnp.int32, (_BATCH, _K), 0)
    q = (bvec * _BEAM + jv) // _LANES
    t = lax.broadcasted_iota(jnp.int32, (_BATCH, _K, _BLK), 2)
    g = ((cv[:, :, None] * _BLK + t) * (_ROWS // _LANES)
         + q[:, :, None])
    g1 = g // 128
    row128 = (g % 128) // 4
    r1 = row128 // 4
    r2 = row128 % 4
    r4 = g % 4
    gidx_ref[...] = ((g1 * 4 + r2) * 8 + r1) * 4 + r4
    lsel = (bvec * _BEAM + jv) % _LANES
    lsel_ref[...] = jnp.broadcast_to(lsel[:, :, None], (_BATCH, _K, _IDXW))


_sc_mesh = plsc.VectorSubcoreMesh(
    core_axis_name="c", subcore_axis_name="s", num_cores=2, num_subcores=16)


@functools.partial(
    pl.kernel,
    out_type=jax.ShapeDtypeStruct((_BATCH * _K, _BLK), jnp.float32),
    mesh=_sc_mesh,
    scratch_types=[
        pltpu.VMEM((2 * _K, _BLK), jnp.int32),
        pltpu.VMEM((2 * _K, _IDXW), jnp.int32),
        pltpu.VMEM((_K, _BLK, _LANES), jnp.float32),
        pltpu.VMEM((2 * _K, _BLK), jnp.float32),
        pltpu.SemaphoreType.DMA,
    ],
    compiler_params=pltpu.CompilerParams(
        use_tc_tiling_on_sc=False, needs_layout_passes=False),
)
def _sc_gather(gidx_hbm, lsel_hbm, table_hbm, out_hbm,
               idx_v, lsel_v, rows_v, comp_v, sem):
    w = lax.axis_index("s") * 2 + lax.axis_index("c")
    pltpu.sync_copy(gidx_hbm.at[pl.ds(w * 2 * _K, 2 * _K)], idx_v)
    pltpu.sync_copy(lsel_hbm.at[pl.ds(w * 2 * _K, 2 * _K)], lsel_v)
    half = _BLK // 2
    for u in range(2):
        base = u * _K
        for grp in range(2):
            copies = []
            for kk in range(8):
                k = grp * 8 + kk
                for h in range(2):
                    copies.append(pltpu.async_copy(
                        table_hbm.at[idx_v.at[base + k, pl.ds(h * half, half)]],
                        rows_v.at[k, pl.ds(h * half, half)], sem))
            for c in copies:
                c.wait()
        for k in range(_K):
            kvec = jnp.full((_IDXW,), k, jnp.int32)
            lanevec = lsel_v[base + k, :]
            for t in range(_BLK // _IDXW):
                tvec = t * _IDXW + lax.iota(jnp.int32, _IDXW)
                vals = plsc.load_gather(rows_v, [kvec, tvec, lanevec])
                comp_v[base + k, pl.ds(t * _IDXW, _IDXW)] = vals
    pltpu.sync_copy(comp_v, out_hbm.at[pl.ds(w * 2 * _K, 2 * _K)])


def _stage_d(cand_ref, ids_ref, lse_ref, sel_ref, sc_ref, wd_ref, bx_ref):
    rows = cand_ref[...]
    ids = ids_ref[...]
    lse = lse_ref[...]
    sel = sel_ref[...]
    jv = ids // _NBLK
    cv = ids - jv * _NBLK
    beams = lax.broadcasted_iota(jnp.int32, (_BATCH, _K, _BEAM), 2)
    onehot = jv[:, :, None] == beams
    lsek = jnp.sum(jnp.where(onehot, lse[:, None, :], 0.0), axis=2)
    selk = jnp.sum(jnp.where(onehot, sel[:, None, :], 0.0), axis=2)
    sc = (rows - lsek[:, :, None]) + selk[:, :, None]
    t = lax.broadcasted_iota(jnp.int32, (_BATCH, _K, _BLK), 2)
    word = cv[:, :, None] * _BLK + t
    sc = jnp.where(word < 3, -jnp.inf, sc)
    fpk = jv[:, :, None] * _VOCAB + word
    scs, pks = [], []
    for _ in range(_K):
        mx = jnp.max(sc, axis=(1, 2), keepdims=True)
        psel = jnp.min(jnp.where(sc == mx, fpk, _BIGI), axis=(1, 2), keepdims=True)
        scs.append(mx[:, :, 0])
        pks.append(psel[:, :, 0])
        sc = jnp.where(fpk == psel, -jnp.inf, sc)
    sc_ref[...] = jnp.concatenate(scs, axis=1)
    pk = jnp.concatenate(pks, axis=1)
    wd_ref[...] = pk % _VOCAB
    bx_ref[...] = pk // _VOCAB


def kernel(logits, scores, position):
    xt = logits.T
    table = (xt.reshape(_TROWS * _LANES // 4096, 8, 4, 128)
             .transpose(0, 2, 1, 3).reshape(_TROWS, _LANES))
    pos = jnp.asarray(position, jnp.int32)
    sel8 = lax.dynamic_index_in_dim(scores, pos - 1, axis=2, keepdims=False)

    bm, lse = pl.pallas_call(
        _stage_a,
        grid=(_STEPS,),
        in_specs=[pl.BlockSpec((_CHUNK, _ROWS), lambda i: (i, 0))],
        out_specs=[
            pl.BlockSpec((1, _SLABS, _ROWS), lambda i: (i, 0, 0)),
            pl.BlockSpec((1, _ROWS), lambda i: (0, 0)),
        ],
        out_shape=[
            jax.ShapeDtypeStruct((_STEPS, _SLABS, _ROWS), jnp.float32),
            jax.ShapeDtypeStruct((1, _ROWS), jnp.float32),
        ],
        scratch_shapes=[
            pltpu.VMEM((1, _ROWS), jnp.float32),
            pltpu.VMEM((1, _ROWS), jnp.float32),
        ],
    )(xt)

    bmt = bm.reshape(_NBLK, _ROWS).T.reshape(_BATCH, _BEAM, _NBLK)
    lse8 = lse.reshape(_ROWS).reshape(_BATCH, _BEAM)

    ids, gidx, lsel = pl.pallas_call(
        _stage_b,
        out_shape=[
            jax.ShapeDtypeStruct((_BATCH, _K), jnp.int32),
            jax.ShapeDtypeStruct((_BATCH, _K, _BLK), jnp.int32),
            jax.ShapeDtypeStruct((_BATCH, _K, _IDXW), jnp.int32),
        ],
    )(bmt, lse8, sel8)

    cand = _sc_gather(gidx.reshape(_BATCH * _K, _BLK),
                      lsel.reshape(_BATCH * _K, _IDXW), table)

    scall, wd, bx = pl.pallas_call(
        _stage_d,
        out_shape=[
            jax.ShapeDtypeStruct((_BATCH, _K), jnp.float32),
            jax.ShapeDtypeStruct((_BATCH, _K), jnp.int32),
            jax.ShapeDtypeStruct((_BATCH, _K), jnp.int32),
        ],
    )(cand.reshape(_BATCH, _K, _BLK), ids, lse8, sel8)

    return (scall[:, :8], wd[:, :8], bx[:, :8], wd, bx)
